# baseline (speedup 1.0000x reference)
import functools

import jax
import jax.numpy as jnp
from jax import lax
from jax.experimental import pallas as pl
from jax.experimental.pallas import tpu as pltpu
from jax.experimental.pallas import tpu_sc as plsc

_DIM = 32
_NX = 4096
_NY = 4096
_PACK = 128 // _DIM

_NC = 2
_NS = 16
_NW = _NC * _NS
_BPW = _NX // _NW
_L = 16


def _gather_body(E4_hbm, xidx_hbm, yidx_hbm, ex_hbm, ey_hbm,
                 xi_v, xr_v, xrows_v, yi_v, yr_v, yrows_v, semx, semy):
    wid = lax.axis_index("s") * _NC + lax.axis_index("c")
    base = wid * _BPW
    pltpu.sync_copy(xidx_hbm.at[pl.ds(base, _BPW)], xi_v)
    pltpu.sync_copy(yidx_hbm.at[pl.ds(base, _BPW)], yi_v)
    for i in range(_BPW // _L):
        s = pl.ds(i * _L, _L)
        xr_v[s] = xi_v[s] >> 2
        yr_v[s] = yi_v[s] >> 2
    cx = pltpu.async_copy(E4_hbm.at[xr_v], xrows_v, semx)
    cy = pltpu.async_copy(E4_hbm.at[yr_v], yrows_v, semy)
    cx.wait()
    cy.wait()
    pltpu.sync_copy(xrows_v, ex_hbm.at[pl.ds(base, _BPW)])
    pltpu.sync_copy(yrows_v, ey_hbm.at[pl.ds(base, _BPW)])


@functools.cache
def _make_gather():
    return functools.partial(
        pl.kernel,
        mesh=plsc.VectorSubcoreMesh(core_axis_name="c", subcore_axis_name="s"),
        out_type=[
            jax.ShapeDtypeStruct((_NX, 128), jnp.float32),
            jax.ShapeDtypeStruct((_NY, 128), jnp.float32),
        ],
        scratch_types=[
            pltpu.VMEM((_BPW,), jnp.int32),
            pltpu.VMEM((_BPW,), jnp.int32),
            pltpu.VMEM((_BPW, 128), jnp.float32),
            pltpu.VMEM((_BPW,), jnp.int32),
            pltpu.VMEM((_BPW,), jnp.int32),
            pltpu.VMEM((_BPW, 128), jnp.float32),
            pltpu.SemaphoreType.DMA,
            pltpu.SemaphoreType.DMA,
        ],
    )(_gather_body)


_BX = 512


def _select_chunk(wide, off):
    acc = jnp.where(off == 0, wide[:, 0:32], 0.0)
    for c in range(1, _PACK):
        acc = acc + jnp.where(off == c, wide[:, c * 32:(c + 1) * 32], 0.0)
    return acc


def _sim_body(xo_ref, yo_ref, exw_ref, eyw_ref, out_ref):
    ex = _select_chunk(exw_ref[...], xo_ref[...] & 3)
    ey = _select_chunk(eyw_ref[...], yo_ref[...] & 3)
    exn = ex / (jnp.sqrt(jnp.sum(ex * ex, axis=1, keepdims=True)) + 1e-8)
    eyn = ey / (jnp.sqrt(jnp.sum(ey * ey, axis=1, keepdims=True)) + 1e-8)
    out_ref[...] = lax.dot_general(
        exn, eyn, (((1,), (1,)), ((), ())),
        preferred_element_type=jnp.float32)


_sim = pl.pallas_call(
    _sim_body,
    grid=(_NX // _BX,),
    in_specs=[
        pl.BlockSpec((_BX, 1), lambda i: (i, 0)),
        pl.BlockSpec((_NY, 1), lambda i: (0, 0)),
        pl.BlockSpec((_BX, 128), lambda i: (i, 0)),
        pl.BlockSpec((_NY, 128), lambda i: (0, 0)),
    ],
    out_specs=pl.BlockSpec((_BX, _NY), lambda i: (i, 0)),
    out_shape=jax.ShapeDtypeStruct((_NX, _NY), jnp.float32),
)


def kernel(x_idx, y_idx, E):
    x_idx = x_idx.astype(jnp.int32)
    y_idx = y_idx.astype(jnp.int32)
    E4 = E.reshape(E.shape[0] // _PACK, 128)
    exw, eyw = _make_gather()(E4, x_idx, y_idx)
    return _sim(x_idx.reshape(_NX, 1), y_idx.reshape(_NY, 1), exw, eyw)
